# Initial kernel scaffold; baseline (speedup 1.0000x reference)
import jax
import jax.numpy as jnp
from jax.experimental import pallas as pl


def kernel(input, target, head_W, w1_0, g_0, b_0, w2_0, w1_1, g_1, b_1, w2_1, w1_2, g_2, b_2, w2_2):
    raise NotImplementedError("write your pallas kernel here")


# speedup vs baseline: 2.1670x; 2.1670x over previous
import jax
import jax.numpy as jnp
from jax.experimental import pallas as pl
from jax.experimental.pallas import tpu as pltpu

_IN = 128
_BATCH = 1024
_NL = 5
_SHORT = 1000
_CUT = [1000, 10000, 40000, 100000]
_CS = [9000, 30000, 60000]
_HSZ = [64, 32, 16]
_HOFF = [0, 64, 96, 112]
_TILE = 1000
_TOFF = [0, 9, 39, 99]
_NT = 99


def _body(x_ref, hw_ref, w1_ref, gb_ref, targ_ref, w20_ref, w21_ref, w22_ref,
          out_ref, stats, hall):
    i = pl.program_id(0)

    @pl.when(i == 0)
    def _prep():
        x = x_ref[...]
        targ = targ_ref[:, 0:_NL]
        ho = jax.lax.dot_general(x, hw_ref[...], (((1,), (1,)), ((), ())),
                                 preferred_element_type=jnp.float32)
        hraw = jax.lax.dot_general(x, w1_ref[...], (((1,), (1,)), ((), ())),
                                   preferred_element_type=jnp.float32)
        g = gb_ref[0:1, :]
        b = gb_ref[1:2, :]
        for ci in range(3):
            lo, hi = _HOFF[ci], _HOFF[ci + 1]
            hseg = hraw[:, lo:hi]
            mu = jnp.mean(hseg, axis=1, keepdims=True)
            var = jnp.mean((hseg - mu) ** 2, axis=1, keepdims=True)
            hn = (hseg - mu) / jnp.sqrt(var + 1e-5) * g[:, lo:hi] + b[:, lo:hi]
            hall[:, lo:hi] = jnp.maximum(hn, 0.0)
        hall[:, _HOFF[3]:_IN] = jnp.zeros((_BATCH, _IN - _HOFF[3]), jnp.float32)

        colid = jax.lax.broadcasted_iota(jnp.int32, (_BATCH, 1024), 1)
        mh = jnp.zeros((_BATCH, 1024), jnp.bool_)
        for k in range(_NL):
            mh = mh | (targ[:, k:k + 1] == colid)
        mh = mh & (colid < _SHORT)
        sp = jnp.maximum(ho, 0.0) + jnp.log1p(jnp.exp(-jnp.abs(ho)))
        maskH = (colid < _SHORT).astype(jnp.float32)
        head_loss = jnp.sum(sp * maskH, axis=1, keepdims=True) \
            - jnp.sum(jnp.where(mh, ho, 0.0), axis=1, keepdims=True)

        num = jnp.full((_BATCH, 1), float(_SHORT), jnp.float32)
        for ci in range(3):
            lo, hi = _CUT[ci], _CUT[ci + 1]
            rm = jnp.zeros((_BATCH, 1), jnp.bool_)
            for k in range(_NL):
                tk = targ[:, k:k + 1]
                rm = rm | ((tk >= lo) & (tk < hi))
            rmf = rm.astype(jnp.float32)
            logit = ho[:, _SHORT + ci:_SHORT + ci + 1]
            sp_r = jnp.maximum(logit, 0.0) + jnp.log1p(jnp.exp(-jnp.abs(logit)))
            head_loss = head_loss + (1.0 - rmf) * sp_r
            num = num + jnp.where(rm, float(_CS[ci]), 1.0)
            stats[:, 2 + ci:3 + ci] = rmf
            stats[:, 5 + ci:6 + ci] = jax.nn.sigmoid(logit)
        stats[:, 0:1] = head_loss
        stats[:, 1:2] = num
        stats[:, 8:9] = jnp.zeros((_BATCH, 1), jnp.float32)

    def tile(ci, w2_ref):
        lo, hi = _HOFF[ci], _HOFF[ci + 1]
        h = hall[:, lo:hi]
        z = jax.lax.dot_general(h, w2_ref[...], (((1,), (1,)), ((), ())),
                                preferred_element_type=jnp.float32)
        r = stats[:, 5 + ci:6 + ci]
        rm = stats[:, 2 + ci:3 + ci]
        base = (i - _TOFF[ci]) * _TILE
        colid = base + jax.lax.broadcasted_iota(jnp.int32, (_BATCH, _TILE), 1)
        tcol = targ_ref[:, 0:_NL] - _CUT[ci]
        m = jnp.zeros((_BATCH, _TILE), jnp.bool_)
        for k in range(_NL):
            m = m | (tcol[:, k:k + 1] == colid)
        p = r * jax.nn.sigmoid(z)
        val = jnp.where(m, p, 1.0 - p)
        contrib = -jnp.maximum(jnp.log(val), -100.0)
        rowsum = jnp.sum(contrib, axis=1, keepdims=True)
        stats[:, 8:9] += rowsum * rm

    @pl.when(i < _TOFF[1])
    def _t0():
        tile(0, w20_ref)

    @pl.when((i >= _TOFF[1]) & (i < _TOFF[2]))
    def _t1():
        tile(1, w21_ref)

    @pl.when(i >= _TOFF[2])
    def _t2():
        tile(2, w22_ref)

    @pl.when(i == _NT - 1)
    def _fin():
        total = (stats[:, 0:1] + stats[:, 8:9]) / stats[:, 1:2]
        out_ref[...] = jnp.full((8, 128), jnp.sum(total) / _BATCH, jnp.float32)


def kernel(input, target, head_W, w1_0, g_0, b_0, w2_0, w1_1, g_1, b_1, w2_1,
           w1_2, g_2, b_2, w2_2):
    f32 = jnp.float32
    hw_pad = jnp.zeros((1024, _IN), f32).at[:head_W.shape[0]].set(head_W)
    w1cat = jnp.zeros((_IN, _IN), f32)
    w1cat = w1cat.at[_HOFF[0]:_HOFF[1]].set(w1_0)
    w1cat = w1cat.at[_HOFF[1]:_HOFF[2]].set(w1_1)
    w1cat = w1cat.at[_HOFF[2]:_HOFF[3]].set(w1_2)
    gb = jnp.zeros((8, _IN), f32)
    gb = gb.at[0, _HOFF[0]:_HOFF[1]].set(g_0).at[1, _HOFF[0]:_HOFF[1]].set(b_0)
    gb = gb.at[0, _HOFF[1]:_HOFF[2]].set(g_1).at[1, _HOFF[1]:_HOFF[2]].set(b_1)
    gb = gb.at[0, _HOFF[2]:_HOFF[3]].set(g_2).at[1, _HOFF[2]:_HOFF[3]].set(b_2)
    targ_pad = jnp.full((_BATCH, 128), -1, jnp.int32).at[:, :_NL].set(target)

    const = lambda i: (0, 0)
    out = pl.pallas_call(
        _body,
        grid=(_NT,),
        in_specs=[
            pl.BlockSpec((_BATCH, _IN), const),
            pl.BlockSpec((1024, _IN), const),
            pl.BlockSpec((_IN, _IN), const),
            pl.BlockSpec((8, _IN), const),
            pl.BlockSpec((_BATCH, 128), const),
            pl.BlockSpec((_TILE, _HSZ[0]), lambda i: (jnp.clip(i, 0, 8), 0)),
            pl.BlockSpec((_TILE, _HSZ[1]), lambda i: (jnp.clip(i - _TOFF[1], 0, 29), 0)),
            pl.BlockSpec((_TILE, _HSZ[2]), lambda i: (jnp.clip(i - _TOFF[2], 0, 59), 0)),
        ],
        out_specs=pl.BlockSpec((8, 128), const),
        out_shape=jax.ShapeDtypeStruct((8, 128), f32),
        scratch_shapes=[
            pltpu.VMEM((_BATCH, 128), f32),
            pltpu.VMEM((_BATCH, _IN), f32),
        ],
    )(input, hw_pad, w1cat, gb, targ_pad, w2_0, w2_1, w2_2)
    return out[0, 0]
